# baseline (speedup 1.0000x reference)
import functools

import jax
import jax.numpy as jnp
from jax import lax
from jax.experimental import pallas as pl
from jax.experimental.pallas import tpu as pltpu
from jax.experimental.pallas import tpu_sc as plsc

NN = 10000
EE = 320000
HH = 128
NLAYER = 4

NP = 10240
NC = 2
NS = 16
NW = NC * NS
CHUNK = 128
NCHUNK = 80
EPW = CHUNK * NCHUNK
EP = NW * EPW
ROWS_PER_TILE = NP // NS
GW = HH + 16

_f32 = jnp.float32


def _silu(x):
    return x * jax.nn.sigmoid(x)


def _sc_mesh():
    return plsc.VectorSubcoreMesh(
        core_axis_name="c", subcore_axis_name="s", num_cores=NC, num_subcores=NS)


def _gather_body(a_h, b_h, xq_h, dst_h, src_h,
                 gd_h, gs_h, xd_h, xs_h,
                 di, si, a0, a1, b0, b1, xd0, xd1, xs0, xs1,
                 semg0, semg1, semw0, semw1):
    cid = lax.axis_index("c")
    sid = lax.axis_index("s")
    wid = sid * NC + cid
    base = wid * EPW
    av = (a0, a1)
    bv = (b0, b1)
    xdv = (xd0, xd1)
    xsv = (xs0, xs1)
    semg = (semg0, semg1)
    semw = (semw0, semw1)

    pltpu.sync_copy(dst_h.at[pl.ds(base, EPW)], di)
    pltpu.sync_copy(src_h.at[pl.ds(base, EPW)], si)

    def gathers(ci, b):
        o = ci * CHUNK
        pltpu.async_copy(a_h.at[di.at[pl.ds(o, CHUNK)]], av[b], semg[b])
        pltpu.async_copy(b_h.at[si.at[pl.ds(o, CHUNK)]], bv[b], semg[b])
        pltpu.async_copy(xq_h.at[di.at[pl.ds(o, CHUNK)]], xdv[b], semg[b])
        pltpu.async_copy(xq_h.at[si.at[pl.ds(o, CHUNK)]], xsv[b], semg[b])

    def drain(sems, b):
        i0 = di.at[pl.ds(0, CHUNK)]
        pltpu.make_async_copy(a_h.at[i0], av[b], sems[b]).wait()
        pltpu.make_async_copy(b_h.at[i0], bv[b], sems[b]).wait()
        pltpu.make_async_copy(xq_h.at[i0], xdv[b], sems[b]).wait()
        pltpu.make_async_copy(xq_h.at[i0], xsv[b], sems[b]).wait()

    def writeback(ci, b):
        o = base + ci * CHUNK
        pltpu.async_copy(av[b], gd_h.at[pl.ds(o, CHUNK)], semw[b])
        pltpu.async_copy(bv[b], gs_h.at[pl.ds(o, CHUNK)], semw[b])
        pltpu.async_copy(xdv[b], xd_h.at[pl.ds(o, CHUNK)], semw[b])
        pltpu.async_copy(xsv[b], xs_h.at[pl.ds(o, CHUNK)], semw[b])

    def outer(g, carry):
        for b in range(2):
            ci = g * 2 + b

            @pl.when(ci >= 2)
            def _():
                drain(semw, b)
            gathers(ci, b)

            @pl.when(ci >= 1)
            def _():
                drain(semg, 1 - b)
                writeback(ci - 1, 1 - b)
        return carry

    lax.fori_loop(0, NCHUNK // 2, outer, 0)
    drain(semg, 1)
    writeback(NCHUNK - 1, 1)
    for b in range(2):
        drain(semw, b)


_bf16 = jnp.bfloat16


def _sc_gather(A, B, XQ, dstp, srcp):
    out_type = (jax.ShapeDtypeStruct((EP, HH), _bf16),
                jax.ShapeDtypeStruct((EP, HH), _bf16),
                jax.ShapeDtypeStruct((EP, 16), _f32),
                jax.ShapeDtypeStruct((EP, 16), _f32))
    scratch = [pltpu.VMEM((EPW,), jnp.int32),
               pltpu.VMEM((EPW,), jnp.int32),
               pltpu.VMEM((CHUNK, HH), _bf16),
               pltpu.VMEM((CHUNK, HH), _bf16),
               pltpu.VMEM((CHUNK, HH), _bf16),
               pltpu.VMEM((CHUNK, HH), _bf16),
               pltpu.VMEM((CHUNK, 16), _f32),
               pltpu.VMEM((CHUNK, 16), _f32),
               pltpu.VMEM((CHUNK, 16), _f32),
               pltpu.VMEM((CHUNK, 16), _f32),
               pltpu.SemaphoreType.DMA,
               pltpu.SemaphoreType.DMA,
               pltpu.SemaphoreType.DMA,
               pltpu.SemaphoreType.DMA]
    fn = pl.kernel(_gather_body, out_type=out_type, mesh=_sc_mesh(),
                   scratch_types=scratch, name="sc_gather",
                   compiler_params=pltpu.CompilerParams(
                       use_tc_tiling_on_sc=False))
    return fn(A, B, XQ, dstp, srcp)


def _zero_vmem(ref, nrow, ncol):
    z = jnp.zeros((16,), _f32)

    def row(r, carry):
        for j in range(ncol // 16):
            ref[r, pl.ds(j * 16, 16)] = z
        return carry

    lax.fori_loop(0, nrow, row, 0)


def _scatter_body(has_t, *refs):
    if has_t:
        (m2_h, t_h, dst_h, magg_h, tacc_h,
         di0, di1, mv0, mv1, tv0, tv1, accm, acct,
         seml0, seml1, sema0, sema1) = refs
    else:
        (m2_h, dst_h, magg_h,
         di0, di1, mv0, mv1, accm,
         seml0, seml1, sema0, sema1) = refs
    cid = lax.axis_index("c")
    sid = lax.axis_index("s")
    wid = sid * NC + cid
    base = wid * EPW
    row0 = sid * ROWS_PER_TILE
    di = (di0, di1)
    mv = (mv0, mv1)
    tv = (tv0, tv1) if has_t else None
    seml = (seml0, seml1)
    sema = (sema0, sema1)

    _zero_vmem(mv0, CHUNK, HH)
    if has_t:
        _zero_vmem(tv0, CHUNK, 16)
    for j in range(ROWS_PER_TILE // CHUNK):
        pltpu.sync_copy(mv0, accm.at[pl.ds(row0 + j * CHUNK, CHUNK)])
        if has_t:
            pltpu.sync_copy(tv0, acct.at[pl.ds(row0 + j * CHUNK, CHUNK)])
    plsc.subcore_barrier()

    def loads(ci, b):
        o = base + ci * CHUNK
        pltpu.async_copy(dst_h.at[pl.ds(o, CHUNK)], di[b], seml[b])
        pltpu.async_copy(m2_h.at[pl.ds(o, CHUNK)], mv[b], seml[b])
        if has_t:
            pltpu.async_copy(t_h.at[pl.ds(o, CHUNK)], tv[b], seml[b])

    def adds(b):
        pltpu.async_copy(mv[b], accm.at[di[b]], sema[b], add=True)
        if has_t:
            pltpu.async_copy(tv[b], acct.at[di[b]], sema[b], add=True)

    def drain_loads(b):
        pltpu.make_async_copy(dst_h.at[pl.ds(0, CHUNK)], di[b],
                              seml[b]).wait()
        pltpu.make_async_copy(m2_h.at[pl.ds(0, CHUNK)], mv[b],
                              seml[b]).wait()
        if has_t:
            pltpu.make_async_copy(t_h.at[pl.ds(0, CHUNK)], tv[b],
                                  seml[b]).wait()

    def drain_adds(b):
        pltpu.make_async_copy(m2_h.at[pl.ds(0, CHUNK)], mv[b],
                              sema[b]).wait()
        if has_t:
            pltpu.make_async_copy(t_h.at[pl.ds(0, CHUNK)], tv[b],
                                  sema[b]).wait()

    def outer(g, carry):
        for b in range(2):
            ci = g * 2 + b

            @pl.when(ci >= 2)
            def _():
                drain_adds(b)
            loads(ci, b)

            @pl.when(ci >= 1)
            def _():
                drain_loads(1 - b)
                adds(1 - b)
        return carry

    lax.fori_loop(0, NCHUNK // 2, outer, 0)
    drain_loads(1)
    adds(1)
    for b in range(2):
        drain_adds(b)
    plsc.subcore_barrier()

    for j in range(ROWS_PER_TILE // CHUNK):
        r = row0 + j * CHUNK
        pltpu.sync_copy(accm.at[pl.ds(r, CHUNK)], magg_h.at[cid, pl.ds(r, CHUNK)])
        if has_t:
            pltpu.sync_copy(acct.at[pl.ds(r, CHUNK)],
                            tacc_h.at[cid, pl.ds(r, CHUNK)])


def _sc_scatter(m2, T, dstp):
    has_t = T is not None
    sems = [pltpu.SemaphoreType.DMA] * 4
    idx = [pltpu.VMEM((CHUNK,), jnp.int32), pltpu.VMEM((CHUNK,), jnp.int32)]
    if has_t:
        out_type = (jax.ShapeDtypeStruct((NC, NP, HH), _f32),
                    jax.ShapeDtypeStruct((NC, NP, 16), _f32))
        scratch = idx + [pltpu.VMEM((CHUNK, HH), _f32),
                         pltpu.VMEM((CHUNK, HH), _f32),
                         pltpu.VMEM((CHUNK, 16), _f32),
                         pltpu.VMEM((CHUNK, 16), _f32),
                         pltpu.VMEM_SHARED((NP, HH), _f32),
                         pltpu.VMEM_SHARED((NP, 16), _f32)] + sems
    else:
        out_type = jax.ShapeDtypeStruct((NC, NP, HH), _f32)
        scratch = idx + [pltpu.VMEM((CHUNK, HH), _f32),
                         pltpu.VMEM((CHUNK, HH), _f32),
                         pltpu.VMEM_SHARED((NP, HH), _f32)] + sems
    fn = pl.kernel(functools.partial(_scatter_body, has_t),
                   out_type=out_type, mesh=_sc_mesh(), scratch_types=scratch,
                   name="sc_scatter",
                   compiler_params=pltpu.CompilerParams(
                       use_tc_tiling_on_sc=False))
    args = (m2, T, dstp) if has_t else (m2, dstp)
    return fn(*args)


BE = 512
BN = 512


def _full(x):
    return pl.BlockSpec(x.shape, lambda i: (0,) * x.ndim)


def _blk(bs):
    nd = len(bs)
    return pl.BlockSpec(bs, lambda i: (i,) + (0,) * (nd - 1))


def _edge_kernel_body(coord, gd, gs, xd, xs, ea, we, w1e, b1, be_, wd2,
                      w2, b2, c1, c1b, c2r, m2_o, t_o=None):
    m_blk = jnp.dot(we[...], w1e[...], preferred_element_type=_f32)
    b1p = b1[...] + jnp.dot(be_[...], w1e[...], preferred_element_type=_f32)
    rel = xd[...] - xs[...]
    d2 = jnp.sum(rel * rel, axis=1, keepdims=True)
    pre = (gd[...].astype(_f32) + gs[...].astype(_f32) + d2 * wd2[...]
           + jnp.dot(ea[...], m_blk, preferred_element_type=_f32) + b1p)
    m = _silu(pre)
    m2 = _silu(jnp.dot(m.astype(_bf16), w2[...].astype(_bf16),
                       preferred_element_type=_f32) + b2[...])
    m2_o[...] = m2
    if coord:
        u2 = _silu(jnp.dot(m2.astype(_bf16), c1[...].astype(_bf16),
                           preferred_element_type=_f32) + c1b[...])
        cw = jnp.sum(u2 * c2r[...], axis=1, keepdims=True)
        lane3 = lax.broadcasted_iota(jnp.int32, (1, 16), 1) == 3
        t_o[...] = rel * cw + lane3.astype(_f32)


def _tc_edge(coord, gd, gs, xd, xs, eap, we, w1e, b1, be_, wd2, w2, b2,
             c1, c1b, c2r):
    grid = EE // BE
    in_specs = [_blk((BE, HH)), _blk((BE, HH)), _blk((BE, 16)), _blk((BE, 16)),
                _blk((BE, 16))] + [_full(w) for w in
                                   (we, w1e, b1, be_, wd2, w2, b2, c1, c1b, c2r)]
    if coord:
        out_shape = (jax.ShapeDtypeStruct((EP, HH), _f32),
                     jax.ShapeDtypeStruct((EP, 16), _f32))
        out_specs = (_blk((BE, HH)), _blk((BE, 16)))
    else:
        out_shape = jax.ShapeDtypeStruct((EP, HH), _f32)
        out_specs = _blk((BE, HH))
    return pl.pallas_call(
        functools.partial(_edge_kernel_body, coord),
        grid=(grid,), in_specs=in_specs, out_specs=out_specs,
        out_shape=out_shape)(gd, gs, xd, xs, eap, we, w1e, b1, be_, wd2,
                             w2, b2, c1, c1b, c2r)


def _node_kernel_body(coord, *refs):
    if coord:
        (h, m0, m1, t0, t1, xq, n1a, n1b, b1n, n2, b2n,
         g, bb, w1d, w1s, h_o, xq_o, a_o, b_o) = refs
    else:
        (h, m0, m1, n1a, n1b, b1n, n2, b2n, g, bb, h_o) = refs
    hv = h[...]
    magg = m0[...][0] + m1[...][0]
    u = _silu(jnp.dot(hv, n1a[...], preferred_element_type=_f32)
              + jnp.dot(magg, n1b[...], preferred_element_type=_f32) + b1n[...])
    hn = hv + jnp.dot(u, n2[...], preferred_element_type=_f32) + b2n[...]
    mu = jnp.mean(hn, axis=1, keepdims=True)
    ctr = hn - mu
    var = jnp.mean(ctr * ctr, axis=1, keepdims=True)
    hln = ctr * jax.lax.rsqrt(var + 1e-5) * g[...] + bb[...]
    h_o[...] = hln
    if coord:
        tacc = t0[...][0] + t1[...][0]
        deg = tacc[:, 3:4]
        invd = 1.0 / jnp.maximum(deg, 1.0)
        lane = lax.broadcasted_iota(jnp.int32, (1, 16), 1)
        xq_o[...] = xq[...] + jnp.where(lane < 3, tacc, 0.0) * invd
        a_o[...] = jnp.dot(hln, w1d[...],
                           preferred_element_type=_f32).astype(_bf16)
        b_o[...] = jnp.dot(hln, w1s[...],
                           preferred_element_type=_f32).astype(_bf16)


def _tc_node(coord, h, magg, tacc, xq, n1a, n1b, b1n, n2, b2n, g, bb,
             w1d, w1s):
    grid = NP // BN
    mspec0 = pl.BlockSpec((1, BN, HH), lambda i: (0, i, 0))
    mspec1 = pl.BlockSpec((1, BN, HH), lambda i: (1, i, 0))
    wspecs = [_full(w) for w in (n1a, n1b, b1n, n2, b2n, g, bb)]
    if coord:
        in_specs = [_blk((BN, HH)), mspec0, mspec1,
                    pl.BlockSpec((1, BN, 16), lambda i: (0, i, 0)),
                    pl.BlockSpec((1, BN, 16), lambda i: (1, i, 0)),
                    _blk((BN, 16))] + wspecs + [_full(w1d), _full(w1s)]
        out_shape = (jax.ShapeDtypeStruct((NP, HH), _f32),
                     jax.ShapeDtypeStruct((NP, 16), _f32),
                     jax.ShapeDtypeStruct((NP, HH), _bf16),
                     jax.ShapeDtypeStruct((NP, HH), _bf16))
        out_specs = (_blk((BN, HH)), _blk((BN, 16)), _blk((BN, HH)),
                     _blk((BN, HH)))
        args = (h, magg, magg, tacc, tacc, xq, n1a, n1b, b1n, n2, b2n,
                g, bb, w1d, w1s)
    else:
        in_specs = [_blk((BN, HH)), mspec0, mspec1] + wspecs
        out_shape = jax.ShapeDtypeStruct((NP, HH), _f32)
        out_specs = _blk((BN, HH))
        args = (h, magg, magg, n1a, n1b, b1n, n2, b2n, g, bb)
    return pl.pallas_call(
        functools.partial(_node_kernel_body, coord),
        grid=(grid,), in_specs=in_specs, out_specs=out_specs,
        out_shape=out_shape)(*args)


def _init_kernel_body(nf, pos, wn, bn, w1d, w1s, h_o, xq_o, a_o, b_o):
    h = (jnp.dot(jnp.clip(nf[...], -100.0, 100.0), wn[...],
                 preferred_element_type=_f32) + bn[...])
    h_o[...] = h
    xq_o[...] = jnp.clip(pos[...], -500.0, 500.0)
    a_o[...] = jnp.dot(h, w1d[...], preferred_element_type=_f32).astype(_bf16)
    b_o[...] = jnp.dot(h, w1s[...], preferred_element_type=_f32).astype(_bf16)


def _tc_init(nfp, pos16, wn, bn, w1d, w1s):
    grid = NP // BN
    in_specs = [_blk((BN, 128)), _blk((BN, 16))] + \
               [_full(w) for w in (wn, bn, w1d, w1s)]
    out_shape = (jax.ShapeDtypeStruct((NP, HH), _f32),
                 jax.ShapeDtypeStruct((NP, 16), _f32),
                 jax.ShapeDtypeStruct((NP, HH), _bf16),
                 jax.ShapeDtypeStruct((NP, HH), _bf16))
    out_specs = (_blk((BN, HH)), _blk((BN, 16)), _blk((BN, HH)),
                 _blk((BN, HH)))
    return pl.pallas_call(
        _init_kernel_body, grid=(grid,), in_specs=in_specs,
        out_specs=out_specs, out_shape=out_shape)(nfp, pos16, wn, bn,
                                                  w1d, w1s)


def kernel(node_features, positions, edge_index, edge_attr, params):
    src = edge_index[0]
    dst = edge_index[1]
    dstp = jnp.pad(dst, (0, EP - EE), constant_values=NN)
    srcp = jnp.pad(src, (0, EP - EE), constant_values=NN)
    nfp = jnp.pad(node_features, ((0, NP - NN), (0, 0)))
    pos16 = jnp.pad(positions, ((0, NP - NN), (0, 13)))

    we = params["edge_embed"]["W"]
    be_ = params["edge_embed"]["b"][None]
    lw = []
    for lp in params["layers"]:
        w1 = lp["edge1"]["W"]
        lw.append(dict(
            w1d=w1[:HH], w1s=w1[HH:2 * HH], wd2=w1[2 * HH:2 * HH + 1],
            w1e=w1[2 * HH + 1:], b1=lp["edge1"]["b"][None],
            w2=lp["edge2"]["W"], b2=lp["edge2"]["b"][None],
            c1=lp["coord1"]["W"], c1b=lp["coord1"]["b"][None],
            c2r=lp["coord2"]["W"].T,
            n1a=lp["node1"]["W"][:HH], n1b=lp["node1"]["W"][HH:],
            b1n=lp["node1"]["b"][None], n2=lp["node2"]["W"],
            b2n=lp["node2"]["b"][None], g=lp["ln_g"][None],
            bb=lp["ln_b"][None]))

    h, xq, A, B = _tc_init(nfp, pos16, params["node_embed"]["W"],
                           params["node_embed"]["b"][None],
                           lw[0]["w1d"], lw[0]["w1s"])

    for i in range(NLAYER):
        w = lw[i]
        coord = i < NLAYER - 1
        GD, GS, XD, XS = _sc_gather(A, B, xq, dstp, srcp)
        if coord:
            m2, T = _tc_edge(True, GD, GS, XD, XS, edge_attr, we, w["w1e"],
                             w["b1"], be_, w["wd2"], w["w2"], w["b2"],
                             w["c1"], w["c1b"], w["c2r"])
            Magg, Tacc = _sc_scatter(m2, T, dstp)
            nx = lw[i + 1]
            h, xq, A, B = _tc_node(
                True, h, Magg, Tacc, xq,
                w["n1a"], w["n1b"], w["b1n"], w["n2"], w["b2n"],
                w["g"], w["bb"], nx["w1d"], nx["w1s"])
        else:
            m2 = _tc_edge(False, GD, GS, XD, XS, edge_attr, we, w["w1e"],
                          w["b1"], be_, w["wd2"], w["w2"], w["b2"],
                          w["c1"], w["c1b"], w["c2r"])
            Magg = _sc_scatter(m2, None, dstp)
            h = _tc_node(False, h, Magg, None, None,
                         w["n1a"], w["n1b"], w["b1n"], w["n2"], w["b2n"],
                         w["g"], w["bb"], None, None)
    return h[:NN]


# speedup vs baseline: 1.1636x; 1.1636x over previous
import functools

import jax
import jax.numpy as jnp
from jax import lax
from jax.experimental import pallas as pl
from jax.experimental.pallas import tpu as pltpu
from jax.experimental.pallas import tpu_sc as plsc

NN = 10000
EE = 320000
HH = 128
NLAYER = 4

NP = 10240
NC = 2
NS = 16
NW = NC * NS
CHUNK = 128
NCHUNK = 80
EPW = CHUNK * NCHUNK
EP = NW * EPW
ROWS_PER_TILE = NP // NS
GW = HH + 16

_f32 = jnp.float32


def _silu(x):
    return x * jax.nn.sigmoid(x)


def _sc_mesh():
    return plsc.VectorSubcoreMesh(
        core_axis_name="c", subcore_axis_name="s", num_cores=NC, num_subcores=NS)


_bf16 = jnp.bfloat16


def _gather2_body(a_h, b_h, dst_h, src_h,
                  gd_h, gs_h,
                  di, si, a0, a1, b0, b1,
                  semg0, semg1, semw0, semw1):
    cid = lax.axis_index("c")
    sid = lax.axis_index("s")
    wid = sid * NC + cid
    base = wid * EPW
    av = (a0, a1)
    bv = (b0, b1)
    semg = (semg0, semg1)
    semw = (semw0, semw1)

    pltpu.sync_copy(dst_h.at[pl.ds(base, EPW)], di)
    pltpu.sync_copy(src_h.at[pl.ds(base, EPW)], si)

    def gathers(ci, b):
        o = ci * CHUNK
        pltpu.async_copy(a_h.at[di.at[pl.ds(o, CHUNK)]], av[b], semg[b])
        pltpu.async_copy(b_h.at[si.at[pl.ds(o, CHUNK)]], bv[b], semg[b])

    def drain(sems, b):
        i0 = di.at[pl.ds(0, CHUNK)]
        pltpu.make_async_copy(a_h.at[i0], av[b], sems[b]).wait()
        pltpu.make_async_copy(b_h.at[i0], bv[b], sems[b]).wait()

    def writeback(ci, b):
        o = base + ci * CHUNK
        pltpu.async_copy(av[b], gd_h.at[pl.ds(o, CHUNK)], semw[b])
        pltpu.async_copy(bv[b], gs_h.at[pl.ds(o, CHUNK)], semw[b])

    def outer(g, carry):
        for b in range(2):
            ci = g * 2 + b

            @pl.when(ci >= 2)
            def _():
                drain(semw, b)
            gathers(ci, b)

            @pl.when(ci >= 1)
            def _():
                drain(semg, 1 - b)
                writeback(ci - 1, 1 - b)
        return carry

    lax.fori_loop(0, NCHUNK // 2, outer, 0)
    drain(semg, 1)
    writeback(NCHUNK - 1, 1)
    for b in range(2):
        drain(semw, b)


def _sc_gather2(A, B, dstp, srcp, width, tc_tiling, name):
    out_type = (jax.ShapeDtypeStruct((EP, width), _f32),
                jax.ShapeDtypeStruct((EP, width), _f32))
    scratch = [pltpu.VMEM((EPW,), jnp.int32),
               pltpu.VMEM((EPW,), jnp.int32),
               pltpu.VMEM((CHUNK, width), _f32),
               pltpu.VMEM((CHUNK, width), _f32),
               pltpu.VMEM((CHUNK, width), _f32),
               pltpu.VMEM((CHUNK, width), _f32),
               pltpu.SemaphoreType.DMA,
               pltpu.SemaphoreType.DMA,
               pltpu.SemaphoreType.DMA,
               pltpu.SemaphoreType.DMA]
    fn = pl.kernel(_gather2_body, out_type=out_type, mesh=_sc_mesh(),
                   scratch_types=scratch, name=name,
                   compiler_params=pltpu.CompilerParams(
                       use_tc_tiling_on_sc=tc_tiling))
    return fn(A, B, dstp, srcp)


def _sc_gather(A, B, XQ, dstp, srcp):
    GD, GS = _sc_gather2(A, B, dstp, srcp, HH, True, "sc_gather_ab")
    XD, XS = _sc_gather2(XQ, XQ, dstp, srcp, 16, False, "sc_gather_x")
    return GD, GS, XD, XS


def _zero_vmem(ref, nrow, ncol):
    z = jnp.zeros((16,), _f32)

    def row(r, carry):
        for j in range(ncol // 16):
            ref[r, pl.ds(j * 16, 16)] = z
        return carry

    lax.fori_loop(0, nrow, row, 0)


def _scatter_body(width, m2_h, dst_h, magg_h,
                  di0, di1, mv0, mv1, accm,
                  seml0, seml1, sema0, sema1):
    cid = lax.axis_index("c")
    sid = lax.axis_index("s")
    wid = sid * NC + cid
    base = wid * EPW
    row0 = sid * ROWS_PER_TILE
    di = (di0, di1)
    mv = (mv0, mv1)
    seml = (seml0, seml1)
    sema = (sema0, sema1)

    _zero_vmem(mv0, CHUNK, width)
    for j in range(ROWS_PER_TILE // CHUNK):
        pltpu.sync_copy(mv0, accm.at[pl.ds(row0 + j * CHUNK, CHUNK)])
    plsc.subcore_barrier()

    def loads(ci, b):
        o = base + ci * CHUNK
        pltpu.async_copy(dst_h.at[pl.ds(o, CHUNK)], di[b], seml[b])
        pltpu.async_copy(m2_h.at[pl.ds(o, CHUNK)], mv[b], seml[b])

    def adds(b):
        pltpu.async_copy(mv[b], accm.at[di[b]], sema[b], add=True)

    def drain_loads(b):
        pltpu.make_async_copy(dst_h.at[pl.ds(0, CHUNK)], di[b],
                              seml[b]).wait()
        pltpu.make_async_copy(m2_h.at[pl.ds(0, CHUNK)], mv[b],
                              seml[b]).wait()

    def drain_adds(b):
        pltpu.make_async_copy(m2_h.at[pl.ds(0, CHUNK)], mv[b],
                              sema[b]).wait()

    def outer(g, carry):
        for b in range(2):
            ci = g * 2 + b

            @pl.when(ci >= 2)
            def _():
                drain_adds(b)
            loads(ci, b)

            @pl.when(ci >= 1)
            def _():
                drain_loads(1 - b)
                adds(1 - b)
        return carry

    lax.fori_loop(0, NCHUNK // 2, outer, 0)
    drain_loads(1)
    adds(1)
    for b in range(2):
        drain_adds(b)
    plsc.subcore_barrier()

    for j in range(ROWS_PER_TILE // CHUNK):
        r = row0 + j * CHUNK
        pltpu.sync_copy(accm.at[pl.ds(r, CHUNK)],
                        magg_h.at[cid, pl.ds(r, CHUNK)])


def _sc_scatter1(m2, dstp, width, tc_tiling, name):
    out_type = jax.ShapeDtypeStruct((NC, NP, width), _f32)
    scratch = [pltpu.VMEM((CHUNK,), jnp.int32),
               pltpu.VMEM((CHUNK,), jnp.int32),
               pltpu.VMEM((CHUNK, width), _f32),
               pltpu.VMEM((CHUNK, width), _f32),
               pltpu.VMEM_SHARED((NP, width), _f32),
               pltpu.SemaphoreType.DMA,
               pltpu.SemaphoreType.DMA,
               pltpu.SemaphoreType.DMA,
               pltpu.SemaphoreType.DMA]
    fn = pl.kernel(functools.partial(_scatter_body, width),
                   out_type=out_type, mesh=_sc_mesh(), scratch_types=scratch,
                   name=name,
                   compiler_params=pltpu.CompilerParams(
                       use_tc_tiling_on_sc=tc_tiling))
    return fn(m2, dstp)


def _sc_scatter(m2, T, dstp):
    Magg = _sc_scatter1(m2, dstp, HH, True, "sc_scatter_m")
    if T is None:
        return Magg
    Tacc = _sc_scatter1(T, dstp, 16, False, "sc_scatter_t")
    return Magg, Tacc


BE = 512
BN = 512


def _full(x):
    return pl.BlockSpec(x.shape, lambda i: (0,) * x.ndim)


def _blk(bs):
    nd = len(bs)
    return pl.BlockSpec(bs, lambda i: (i,) + (0,) * (nd - 1))


def _edge_kernel_body(coord, gd, gs, xd, xs, ea, we, w1e, b1, be_, wd2,
                      w2, b2, c1, c1b, c2r, m2_o, t_o=None):
    m_blk = jnp.dot(we[...], w1e[...], preferred_element_type=_f32)
    b1p = b1[...] + jnp.dot(be_[...], w1e[...], preferred_element_type=_f32)
    rel = xd[...] - xs[...]
    d2 = jnp.sum(rel * rel, axis=1, keepdims=True)
    pre = (gd[...].astype(_f32) + gs[...].astype(_f32) + d2 * wd2[...]
           + jnp.dot(ea[...], m_blk, preferred_element_type=_f32) + b1p)
    m = _silu(pre)
    m2 = _silu(jnp.dot(m.astype(_bf16), w2[...].astype(_bf16),
                       preferred_element_type=_f32) + b2[...])
    m2_o[...] = m2
    if coord:
        u2 = _silu(jnp.dot(m2.astype(_bf16), c1[...].astype(_bf16),
                           preferred_element_type=_f32) + c1b[...])
        cw = jnp.sum(u2 * c2r[...], axis=1, keepdims=True)
        lane3 = lax.broadcasted_iota(jnp.int32, (1, 16), 1) == 3
        t_o[...] = rel * cw + lane3.astype(_f32)


def _tc_edge(coord, gd, gs, xd, xs, eap, we, w1e, b1, be_, wd2, w2, b2,
             c1, c1b, c2r):
    grid = EE // BE
    in_specs = [_blk((BE, HH)), _blk((BE, HH)), _blk((BE, 16)), _blk((BE, 16)),
                _blk((BE, 16))] + [_full(w) for w in
                                   (we, w1e, b1, be_, wd2, w2, b2, c1, c1b, c2r)]
    if coord:
        out_shape = (jax.ShapeDtypeStruct((EP, HH), _f32),
                     jax.ShapeDtypeStruct((EP, 16), _f32))
        out_specs = (_blk((BE, HH)), _blk((BE, 16)))
    else:
        out_shape = jax.ShapeDtypeStruct((EP, HH), _f32)
        out_specs = _blk((BE, HH))
    return pl.pallas_call(
        functools.partial(_edge_kernel_body, coord),
        grid=(grid,), in_specs=in_specs, out_specs=out_specs,
        out_shape=out_shape)(gd, gs, xd, xs, eap, we, w1e, b1, be_, wd2,
                             w2, b2, c1, c1b, c2r)


def _node_kernel_body(coord, *refs):
    if coord:
        (h, m0, m1, t0, t1, xq, n1a, n1b, b1n, n2, b2n,
         g, bb, w1d, w1s, h_o, xq_o, a_o, b_o) = refs
    else:
        (h, m0, m1, n1a, n1b, b1n, n2, b2n, g, bb, h_o) = refs
    hv = h[...]
    magg = m0[...][0] + m1[...][0]
    u = _silu(jnp.dot(hv, n1a[...], preferred_element_type=_f32)
              + jnp.dot(magg, n1b[...], preferred_element_type=_f32) + b1n[...])
    hn = hv + jnp.dot(u, n2[...], preferred_element_type=_f32) + b2n[...]
    mu = jnp.mean(hn, axis=1, keepdims=True)
    ctr = hn - mu
    var = jnp.mean(ctr * ctr, axis=1, keepdims=True)
    hln = ctr * jax.lax.rsqrt(var + 1e-5) * g[...] + bb[...]
    h_o[...] = hln
    if coord:
        tacc = t0[...][0] + t1[...][0]
        deg = tacc[:, 3:4]
        invd = 1.0 / jnp.maximum(deg, 1.0)
        lane = lax.broadcasted_iota(jnp.int32, (1, 16), 1)
        xq_o[...] = xq[...] + jnp.where(lane < 3, tacc, 0.0) * invd
        a_o[...] = jnp.dot(hln, w1d[...], preferred_element_type=_f32)
        b_o[...] = jnp.dot(hln, w1s[...], preferred_element_type=_f32)


def _tc_node(coord, h, magg, tacc, xq, n1a, n1b, b1n, n2, b2n, g, bb,
             w1d, w1s):
    grid = NP // BN
    mspec0 = pl.BlockSpec((1, BN, HH), lambda i: (0, i, 0))
    mspec1 = pl.BlockSpec((1, BN, HH), lambda i: (1, i, 0))
    wspecs = [_full(w) for w in (n1a, n1b, b1n, n2, b2n, g, bb)]
    if coord:
        in_specs = [_blk((BN, HH)), mspec0, mspec1,
                    pl.BlockSpec((1, BN, 16), lambda i: (0, i, 0)),
                    pl.BlockSpec((1, BN, 16), lambda i: (1, i, 0)),
                    _blk((BN, 16))] + wspecs + [_full(w1d), _full(w1s)]
        out_shape = (jax.ShapeDtypeStruct((NP, HH), _f32),
                     jax.ShapeDtypeStruct((NP, 16), _f32),
                     jax.ShapeDtypeStruct((NP, HH), _f32),
                     jax.ShapeDtypeStruct((NP, HH), _f32))
        out_specs = (_blk((BN, HH)), _blk((BN, 16)), _blk((BN, HH)),
                     _blk((BN, HH)))
        args = (h, magg, magg, tacc, tacc, xq, n1a, n1b, b1n, n2, b2n,
                g, bb, w1d, w1s)
    else:
        in_specs = [_blk((BN, HH)), mspec0, mspec1] + wspecs
        out_shape = jax.ShapeDtypeStruct((NP, HH), _f32)
        out_specs = _blk((BN, HH))
        args = (h, magg, magg, n1a, n1b, b1n, n2, b2n, g, bb)
    return pl.pallas_call(
        functools.partial(_node_kernel_body, coord),
        grid=(grid,), in_specs=in_specs, out_specs=out_specs,
        out_shape=out_shape)(*args)


def _init_kernel_body(nf, pos, wn, bn, w1d, w1s, h_o, xq_o, a_o, b_o):
    h = (jnp.dot(jnp.clip(nf[...], -100.0, 100.0), wn[...],
                 preferred_element_type=_f32) + bn[...])
    h_o[...] = h
    xq_o[...] = jnp.clip(pos[...], -500.0, 500.0)
    a_o[...] = jnp.dot(h, w1d[...], preferred_element_type=_f32)
    b_o[...] = jnp.dot(h, w1s[...], preferred_element_type=_f32)


def _tc_init(nfp, pos16, wn, bn, w1d, w1s):
    grid = NP // BN
    in_specs = [_blk((BN, 128)), _blk((BN, 16))] + \
               [_full(w) for w in (wn, bn, w1d, w1s)]
    out_shape = (jax.ShapeDtypeStruct((NP, HH), _f32),
                 jax.ShapeDtypeStruct((NP, 16), _f32),
                 jax.ShapeDtypeStruct((NP, HH), _f32),
                 jax.ShapeDtypeStruct((NP, HH), _f32))
    out_specs = (_blk((BN, HH)), _blk((BN, 16)), _blk((BN, HH)),
                 _blk((BN, HH)))
    return pl.pallas_call(
        _init_kernel_body, grid=(grid,), in_specs=in_specs,
        out_specs=out_specs, out_shape=out_shape)(nfp, pos16, wn, bn,
                                                  w1d, w1s)


def kernel(node_features, positions, edge_index, edge_attr, params):
    src = edge_index[0]
    dst = edge_index[1]
    dstp = jnp.pad(dst, (0, EP - EE), constant_values=NN)
    srcp = jnp.pad(src, (0, EP - EE), constant_values=NN)
    nfp = jnp.pad(node_features, ((0, NP - NN), (0, 0)))
    pos16 = jnp.pad(positions, ((0, NP - NN), (0, 13)))

    we = params["edge_embed"]["W"]
    be_ = params["edge_embed"]["b"][None]
    lw = []
    for lp in params["layers"]:
        w1 = lp["edge1"]["W"]
        lw.append(dict(
            w1d=w1[:HH], w1s=w1[HH:2 * HH], wd2=w1[2 * HH:2 * HH + 1],
            w1e=w1[2 * HH + 1:], b1=lp["edge1"]["b"][None],
            w2=lp["edge2"]["W"], b2=lp["edge2"]["b"][None],
            c1=lp["coord1"]["W"], c1b=lp["coord1"]["b"][None],
            c2r=lp["coord2"]["W"].T,
            n1a=lp["node1"]["W"][:HH], n1b=lp["node1"]["W"][HH:],
            b1n=lp["node1"]["b"][None], n2=lp["node2"]["W"],
            b2n=lp["node2"]["b"][None], g=lp["ln_g"][None],
            bb=lp["ln_b"][None]))

    h, xq, A, B = _tc_init(nfp, pos16, params["node_embed"]["W"],
                           params["node_embed"]["b"][None],
                           lw[0]["w1d"], lw[0]["w1s"])

    for i in range(NLAYER):
        w = lw[i]
        coord = i < NLAYER - 1
        GD, GS, XD, XS = _sc_gather(A, B, xq, dstp, srcp)
        if coord:
            m2, T = _tc_edge(True, GD, GS, XD, XS, edge_attr, we, w["w1e"],
                             w["b1"], be_, w["wd2"], w["w2"], w["b2"],
                             w["c1"], w["c1b"], w["c2r"])
            Magg, Tacc = _sc_scatter(m2, T, dstp)
            nx = lw[i + 1]
            h, xq, A, B = _tc_node(
                True, h, Magg, Tacc, xq,
                w["n1a"], w["n1b"], w["b1n"], w["n2"], w["b2n"],
                w["g"], w["bb"], nx["w1d"], nx["w1s"])
        else:
            m2 = _tc_edge(False, GD, GS, XD, XS, edge_attr, we, w["w1e"],
                          w["b1"], be_, w["wd2"], w["w2"], w["b2"],
                          w["c1"], w["c1b"], w["c2r"])
            Magg = _sc_scatter(m2, None, dstp)
            h = _tc_node(False, h, Magg, None, None,
                         w["n1a"], w["n1b"], w["b1n"], w["n2"], w["b2n"],
                         w["g"], w["bb"], None, None)
    return h[:NN]


# speedup vs baseline: 1.2496x; 1.0739x over previous
import functools

import jax
import jax.numpy as jnp
from jax import lax
from jax.experimental import pallas as pl
from jax.experimental.pallas import tpu as pltpu
from jax.experimental.pallas import tpu_sc as plsc

NN = 10000
EE = 320000
HH = 128
NLAYER = 4

NP = 10240
NC = 2
NS = 16
NW = NC * NS
CHUNK = 128
NCHUNK = 80
EPW = CHUNK * NCHUNK
EP = NW * EPW
ROWS_PER_TILE = NP // NS
GW = HH + 16

_f32 = jnp.float32


def _silu(x):
    return x * jax.nn.sigmoid(x)


def _sc_mesh():
    return plsc.VectorSubcoreMesh(
        core_axis_name="c", subcore_axis_name="s", num_cores=NC, num_subcores=NS)


_bf16 = jnp.bfloat16


def _gather2_body(width, presum, *refs):
    if presum:
        (a_h, b_h, dst_h, src_h, g_h,
         di, si, a0, a1, b0, b1, semg0, semg1, semw0, semw1) = refs
    else:
        (a_h, b_h, dst_h, src_h, gd_h, gs_h,
         di, si, a0, a1, b0, b1, semg0, semg1, semw0, semw1) = refs
    cid = lax.axis_index("c")
    sid = lax.axis_index("s")
    wid = sid * NC + cid
    base = wid * EPW
    av = (a0, a1)
    bv = (b0, b1)
    semg = (semg0, semg1)
    semw = (semw0, semw1)

    pltpu.sync_copy(dst_h.at[pl.ds(base, EPW)], di)
    pltpu.sync_copy(src_h.at[pl.ds(base, EPW)], si)

    def gathers(ci, b):
        o = ci * CHUNK
        pltpu.async_copy(a_h.at[di.at[pl.ds(o, CHUNK)]], av[b], semg[b])
        pltpu.async_copy(b_h.at[si.at[pl.ds(o, CHUNK)]], bv[b], semg[b])

    def drain(sems, b):
        i0 = di.at[pl.ds(0, CHUNK)]
        pltpu.make_async_copy(a_h.at[i0], av[b], sems[b]).wait()
        pltpu.make_async_copy(b_h.at[i0], bv[b], sems[b]).wait()

    def drainw(b):
        i0 = di.at[pl.ds(0, CHUNK)]
        pltpu.make_async_copy(a_h.at[i0], av[b], semw[b]).wait()
        if not presum:
            pltpu.make_async_copy(b_h.at[i0], bv[b], semw[b]).wait()

    def writeback(ci, b):
        o = base + ci * CHUNK
        if presum:
            def row(r, carry):
                for j in range(width // 16):
                    s = pl.ds(j * 16, 16)
                    av[b][r, s] = av[b][r, s] + bv[b][r, s]
                return carry

            lax.fori_loop(0, CHUNK, row, 0)
            pltpu.async_copy(av[b], g_h.at[pl.ds(o, CHUNK)], semw[b])
        else:
            pltpu.async_copy(av[b], gd_h.at[pl.ds(o, CHUNK)], semw[b])
            pltpu.async_copy(bv[b], gs_h.at[pl.ds(o, CHUNK)], semw[b])

    def outer(g, carry):
        for b in range(2):
            ci = g * 2 + b

            @pl.when(ci >= 2)
            def _():
                drainw(b)
            gathers(ci, b)

            @pl.when(ci >= 1)
            def _():
                drain(semg, 1 - b)
                writeback(ci - 1, 1 - b)
        return carry

    lax.fori_loop(0, NCHUNK // 2, outer, 0)
    drain(semg, 1)
    writeback(NCHUNK - 1, 1)
    for b in range(2):
        drainw(b)


def _sc_gather2(A, B, dstp, srcp, width, tc_tiling, presum, name):
    if presum:
        out_type = jax.ShapeDtypeStruct((EP, width), _f32)
    else:
        out_type = (jax.ShapeDtypeStruct((EP, width), _f32),
                    jax.ShapeDtypeStruct((EP, width), _f32))
    scratch = [pltpu.VMEM((EPW,), jnp.int32),
               pltpu.VMEM((EPW,), jnp.int32),
               pltpu.VMEM((CHUNK, width), _f32),
               pltpu.VMEM((CHUNK, width), _f32),
               pltpu.VMEM((CHUNK, width), _f32),
               pltpu.VMEM((CHUNK, width), _f32),
               pltpu.SemaphoreType.DMA,
               pltpu.SemaphoreType.DMA,
               pltpu.SemaphoreType.DMA,
               pltpu.SemaphoreType.DMA]
    fn = pl.kernel(functools.partial(_gather2_body, width, presum),
                   out_type=out_type, mesh=_sc_mesh(),
                   scratch_types=scratch, name=name,
                   compiler_params=pltpu.CompilerParams(
                       use_tc_tiling_on_sc=tc_tiling))
    return fn(A, B, dstp, srcp)


def _sc_gather(A, B, XQ, dstp, srcp):
    G = _sc_gather2(A, B, dstp, srcp, HH, True, True, "sc_gather_ab")
    XD, XS = _sc_gather2(XQ, XQ, dstp, srcp, 16, False, False, "sc_gather_x")
    return G, XD, XS


def _zero_vmem(ref, nrow, ncol):
    z = jnp.zeros((16,), _f32)

    def row(r, carry):
        for j in range(ncol // 16):
            ref[r, pl.ds(j * 16, 16)] = z
        return carry

    lax.fori_loop(0, nrow, row, 0)


def _scatter_body(width, m2_h, dst_h, magg_h,
                  di0, di1, mv0, mv1, accm,
                  seml0, seml1, sema0, sema1):
    cid = lax.axis_index("c")
    sid = lax.axis_index("s")
    wid = sid * NC + cid
    base = wid * EPW
    row0 = sid * ROWS_PER_TILE
    di = (di0, di1)
    mv = (mv0, mv1)
    seml = (seml0, seml1)
    sema = (sema0, sema1)

    _zero_vmem(mv0, CHUNK, width)
    for j in range(ROWS_PER_TILE // CHUNK):
        pltpu.sync_copy(mv0, accm.at[pl.ds(row0 + j * CHUNK, CHUNK)])
    plsc.subcore_barrier()

    def loads(ci, b):
        o = base + ci * CHUNK
        pltpu.async_copy(dst_h.at[pl.ds(o, CHUNK)], di[b], seml[b])
        pltpu.async_copy(m2_h.at[pl.ds(o, CHUNK)], mv[b], seml[b])

    def adds(b):
        pltpu.async_copy(mv[b], accm.at[di[b]], sema[b], add=True)

    def drain_loads(b):
        pltpu.make_async_copy(dst_h.at[pl.ds(0, CHUNK)], di[b],
                              seml[b]).wait()
        pltpu.make_async_copy(m2_h.at[pl.ds(0, CHUNK)], mv[b],
                              seml[b]).wait()

    def drain_adds(b):
        pltpu.make_async_copy(m2_h.at[pl.ds(0, CHUNK)], mv[b],
                              sema[b]).wait()

    def outer(g, carry):
        for b in range(2):
            ci = g * 2 + b

            @pl.when(ci >= 2)
            def _():
                drain_adds(b)
            loads(ci, b)

            @pl.when(ci >= 1)
            def _():
                drain_loads(1 - b)
                adds(1 - b)
        return carry

    lax.fori_loop(0, NCHUNK // 2, outer, 0)
    drain_loads(1)
    adds(1)
    for b in range(2):
        drain_adds(b)
    plsc.subcore_barrier()

    for j in range(ROWS_PER_TILE // CHUNK):
        r = row0 + j * CHUNK
        pltpu.sync_copy(accm.at[pl.ds(r, CHUNK)],
                        magg_h.at[cid, pl.ds(r, CHUNK)])


def _sc_scatter1(m2, dstp, width, tc_tiling, name):
    out_type = jax.ShapeDtypeStruct((NC, NP, width), _f32)
    scratch = [pltpu.VMEM((CHUNK,), jnp.int32),
               pltpu.VMEM((CHUNK,), jnp.int32),
               pltpu.VMEM((CHUNK, width), _f32),
               pltpu.VMEM((CHUNK, width), _f32),
               pltpu.VMEM_SHARED((NP, width), _f32),
               pltpu.SemaphoreType.DMA,
               pltpu.SemaphoreType.DMA,
               pltpu.SemaphoreType.DMA,
               pltpu.SemaphoreType.DMA]
    fn = pl.kernel(functools.partial(_scatter_body, width),
                   out_type=out_type, mesh=_sc_mesh(), scratch_types=scratch,
                   name=name,
                   compiler_params=pltpu.CompilerParams(
                       use_tc_tiling_on_sc=tc_tiling))
    return fn(m2, dstp)


def _sc_scatter(m2, T, dstp):
    Magg = _sc_scatter1(m2, dstp, HH, True, "sc_scatter_m")
    if T is None:
        return Magg
    Tacc = _sc_scatter1(T, dstp, 16, False, "sc_scatter_t")
    return Magg, Tacc


BE = 512
BN = 512


def _full(x):
    return pl.BlockSpec(x.shape, lambda i: (0,) * x.ndim)


def _blk(bs):
    nd = len(bs)
    return pl.BlockSpec(bs, lambda i: (i,) + (0,) * (nd - 1))


def _edge_kernel_body(coord, g, xd, xs, ea, we, w1e, b1, be_, wd2,
                      w2, b2, c1, c1b, c2r, m2_o, t_o=None):
    m_blk = jnp.dot(we[...], w1e[...], preferred_element_type=_f32)
    b1p = b1[...] + jnp.dot(be_[...], w1e[...], preferred_element_type=_f32)
    rel = xd[...] - xs[...]
    d2 = jnp.sum(rel * rel, axis=1, keepdims=True)
    pre = (g[...] + d2 * wd2[...]
           + jnp.dot(ea[...], m_blk, preferred_element_type=_f32) + b1p)
    m = _silu(pre)
    m2 = _silu(jnp.dot(m.astype(_bf16), w2[...].astype(_bf16),
                       preferred_element_type=_f32) + b2[...])
    m2_o[...] = m2
    if coord:
        u2 = _silu(jnp.dot(m2.astype(_bf16), c1[...].astype(_bf16),
                           preferred_element_type=_f32) + c1b[...])
        cw = jnp.sum(u2 * c2r[...], axis=1, keepdims=True)
        lane3 = lax.broadcasted_iota(jnp.int32, (1, 16), 1) == 3
        t_o[...] = rel * cw + lane3.astype(_f32)


def _tc_edge(coord, g, xd, xs, eap, we, w1e, b1, be_, wd2, w2, b2,
             c1, c1b, c2r):
    grid = EE // BE
    in_specs = [_blk((BE, HH)), _blk((BE, 16)), _blk((BE, 16)),
                _blk((BE, 16))] + [_full(w) for w in
                                   (we, w1e, b1, be_, wd2, w2, b2, c1, c1b, c2r)]
    if coord:
        out_shape = (jax.ShapeDtypeStruct((EP, HH), _f32),
                     jax.ShapeDtypeStruct((EP, 16), _f32))
        out_specs = (_blk((BE, HH)), _blk((BE, 16)))
    else:
        out_shape = jax.ShapeDtypeStruct((EP, HH), _f32)
        out_specs = _blk((BE, HH))
    return pl.pallas_call(
        functools.partial(_edge_kernel_body, coord),
        grid=(grid,), in_specs=in_specs, out_specs=out_specs,
        out_shape=out_shape)(g, xd, xs, eap, we, w1e, b1, be_, wd2,
                             w2, b2, c1, c1b, c2r)


def _node_kernel_body(coord, *refs):
    if coord:
        (h, m0, m1, t0, t1, xq, n1a, n1b, b1n, n2, b2n,
         g, bb, w1d, w1s, h_o, xq_o, a_o, b_o) = refs
    else:
        (h, m0, m1, n1a, n1b, b1n, n2, b2n, g, bb, h_o) = refs
    hv = h[...]
    magg = m0[...][0] + m1[...][0]
    u = _silu(jnp.dot(hv, n1a[...], preferred_element_type=_f32)
              + jnp.dot(magg, n1b[...], preferred_element_type=_f32) + b1n[...])
    hn = hv + jnp.dot(u, n2[...], preferred_element_type=_f32) + b2n[...]
    mu = jnp.mean(hn, axis=1, keepdims=True)
    ctr = hn - mu
    var = jnp.mean(ctr * ctr, axis=1, keepdims=True)
    hln = ctr * jax.lax.rsqrt(var + 1e-5) * g[...] + bb[...]
    h_o[...] = hln
    if coord:
        tacc = t0[...][0] + t1[...][0]
        deg = tacc[:, 3:4]
        invd = 1.0 / jnp.maximum(deg, 1.0)
        lane = lax.broadcasted_iota(jnp.int32, (1, 16), 1)
        xq_o[...] = xq[...] + jnp.where(lane < 3, tacc, 0.0) * invd
        a_o[...] = jnp.dot(hln, w1d[...], preferred_element_type=_f32)
        b_o[...] = jnp.dot(hln, w1s[...], preferred_element_type=_f32)


def _tc_node(coord, h, magg, tacc, xq, n1a, n1b, b1n, n2, b2n, g, bb,
             w1d, w1s):
    grid = NP // BN
    mspec0 = pl.BlockSpec((1, BN, HH), lambda i: (0, i, 0))
    mspec1 = pl.BlockSpec((1, BN, HH), lambda i: (1, i, 0))
    wspecs = [_full(w) for w in (n1a, n1b, b1n, n2, b2n, g, bb)]
    if coord:
        in_specs = [_blk((BN, HH)), mspec0, mspec1,
                    pl.BlockSpec((1, BN, 16), lambda i: (0, i, 0)),
                    pl.BlockSpec((1, BN, 16), lambda i: (1, i, 0)),
                    _blk((BN, 16))] + wspecs + [_full(w1d), _full(w1s)]
        out_shape = (jax.ShapeDtypeStruct((NP, HH), _f32),
                     jax.ShapeDtypeStruct((NP, 16), _f32),
                     jax.ShapeDtypeStruct((NP, HH), _f32),
                     jax.ShapeDtypeStruct((NP, HH), _f32))
        out_specs = (_blk((BN, HH)), _blk((BN, 16)), _blk((BN, HH)),
                     _blk((BN, HH)))
        args = (h, magg, magg, tacc, tacc, xq, n1a, n1b, b1n, n2, b2n,
                g, bb, w1d, w1s)
    else:
        in_specs = [_blk((BN, HH)), mspec0, mspec1] + wspecs
        out_shape = jax.ShapeDtypeStruct((NP, HH), _f32)
        out_specs = _blk((BN, HH))
        args = (h, magg, magg, n1a, n1b, b1n, n2, b2n, g, bb)
    return pl.pallas_call(
        functools.partial(_node_kernel_body, coord),
        grid=(grid,), in_specs=in_specs, out_specs=out_specs,
        out_shape=out_shape)(*args)


def _init_kernel_body(nf, pos, wn, bn, w1d, w1s, h_o, xq_o, a_o, b_o):
    h = (jnp.dot(jnp.clip(nf[...], -100.0, 100.0), wn[...],
                 preferred_element_type=_f32) + bn[...])
    h_o[...] = h
    xq_o[...] = jnp.clip(pos[...], -500.0, 500.0)
    a_o[...] = jnp.dot(h, w1d[...], preferred_element_type=_f32)
    b_o[...] = jnp.dot(h, w1s[...], preferred_element_type=_f32)


def _tc_init(nfp, pos16, wn, bn, w1d, w1s):
    grid = NP // BN
    in_specs = [_blk((BN, 128)), _blk((BN, 16))] + \
               [_full(w) for w in (wn, bn, w1d, w1s)]
    out_shape = (jax.ShapeDtypeStruct((NP, HH), _f32),
                 jax.ShapeDtypeStruct((NP, 16), _f32),
                 jax.ShapeDtypeStruct((NP, HH), _f32),
                 jax.ShapeDtypeStruct((NP, HH), _f32))
    out_specs = (_blk((BN, HH)), _blk((BN, 16)), _blk((BN, HH)),
                 _blk((BN, HH)))
    return pl.pallas_call(
        _init_kernel_body, grid=(grid,), in_specs=in_specs,
        out_specs=out_specs, out_shape=out_shape)(nfp, pos16, wn, bn,
                                                  w1d, w1s)


def kernel(node_features, positions, edge_index, edge_attr, params):
    src = edge_index[0]
    dst = edge_index[1]
    dstp = jnp.pad(dst, (0, EP - EE), constant_values=NN)
    srcp = jnp.pad(src, (0, EP - EE), constant_values=NN)
    nfp = jnp.pad(node_features, ((0, NP - NN), (0, 0)))
    pos16 = jnp.pad(positions, ((0, NP - NN), (0, 13)))

    we = params["edge_embed"]["W"]
    be_ = params["edge_embed"]["b"][None]
    lw = []
    for lp in params["layers"]:
        w1 = lp["edge1"]["W"]
        lw.append(dict(
            w1d=w1[:HH], w1s=w1[HH:2 * HH], wd2=w1[2 * HH:2 * HH + 1],
            w1e=w1[2 * HH + 1:], b1=lp["edge1"]["b"][None],
            w2=lp["edge2"]["W"], b2=lp["edge2"]["b"][None],
            c1=lp["coord1"]["W"], c1b=lp["coord1"]["b"][None],
            c2r=lp["coord2"]["W"].T,
            n1a=lp["node1"]["W"][:HH], n1b=lp["node1"]["W"][HH:],
            b1n=lp["node1"]["b"][None], n2=lp["node2"]["W"],
            b2n=lp["node2"]["b"][None], g=lp["ln_g"][None],
            bb=lp["ln_b"][None]))

    h, xq, A, B = _tc_init(nfp, pos16, params["node_embed"]["W"],
                           params["node_embed"]["b"][None],
                           lw[0]["w1d"], lw[0]["w1s"])

    for i in range(NLAYER):
        w = lw[i]
        coord = i < NLAYER - 1
        G, XD, XS = _sc_gather(A, B, xq, dstp, srcp)
        if coord:
            m2, T = _tc_edge(True, G, XD, XS, edge_attr, we, w["w1e"],
                             w["b1"], be_, w["wd2"], w["w2"], w["b2"],
                             w["c1"], w["c1b"], w["c2r"])
            Magg, Tacc = _sc_scatter(m2, T, dstp)
            nx = lw[i + 1]
            h, xq, A, B = _tc_node(
                True, h, Magg, Tacc, xq,
                w["n1a"], w["n1b"], w["b1n"], w["n2"], w["b2n"],
                w["g"], w["bb"], nx["w1d"], nx["w1s"])
        else:
            m2 = _tc_edge(False, G, XD, XS, edge_attr, we, w["w1e"],
                          w["b1"], be_, w["wd2"], w["w2"], w["b2"],
                          w["c1"], w["c1b"], w["c2r"])
            Magg = _sc_scatter(m2, None, dstp)
            h = _tc_node(False, h, Magg, None, None,
                         w["n1a"], w["n1b"], w["b1n"], w["n2"], w["b2n"],
                         w["g"], w["bb"], None, None)
    return h[:NN]


# speedup vs baseline: 1.4228x; 1.1385x over previous
import functools

import jax
import jax.numpy as jnp
from jax import lax
from jax.experimental import pallas as pl
from jax.experimental.pallas import tpu as pltpu
from jax.experimental.pallas import tpu_sc as plsc

NN = 10000
EE = 320000
HH = 128
NLAYER = 4

NP = 10240
NC = 2
NS = 16
NW = NC * NS
CHUNK = 128
NCHUNK = 80
EPW = CHUNK * NCHUNK
EP = NW * EPW
ROWS_PER_TILE = NP // NS
NHALF = 2
EP2 = EP // NHALF
EPW_H = EPW // NHALF
NCHUNK_H = NCHUNK // NHALF
GW = HH + 16

_f32 = jnp.float32


def _silu(x):
    return x * jax.nn.sigmoid(x)


def _sc_mesh():
    return plsc.VectorSubcoreMesh(
        core_axis_name="c", subcore_axis_name="s", num_cores=NC, num_subcores=NS)


_bf16 = jnp.bfloat16


def _gather2_body(width, presum, half, *refs):
    if presum:
        (a_h, b_h, dst_h, src_h, g_h,
         di, si, a0, a1, b0, b1, semg0, semg1, semw0, semw1) = refs
    else:
        (a_h, b_h, dst_h, src_h, gd_h, gs_h,
         di, si, a0, a1, b0, b1, semg0, semg1, semw0, semw1) = refs
    cid = lax.axis_index("c")
    sid = lax.axis_index("s")
    wid = sid * NC + cid
    base = wid * EPW_H
    gbase = half * EP2 + base
    av = (a0, a1)
    bv = (b0, b1)
    semg = (semg0, semg1)
    semw = (semw0, semw1)

    pltpu.sync_copy(dst_h.at[pl.ds(gbase, EPW_H)], di)
    pltpu.sync_copy(src_h.at[pl.ds(gbase, EPW_H)], si)

    def gathers(ci, b):
        o = ci * CHUNK
        pltpu.async_copy(a_h.at[di.at[pl.ds(o, CHUNK)]], av[b], semg[b])
        pltpu.async_copy(b_h.at[si.at[pl.ds(o, CHUNK)]], bv[b], semg[b])

    def drain(sems, b):
        i0 = di.at[pl.ds(0, CHUNK)]
        pltpu.make_async_copy(a_h.at[i0], av[b], sems[b]).wait()
        pltpu.make_async_copy(b_h.at[i0], bv[b], sems[b]).wait()

    def drainw(b):
        i0 = di.at[pl.ds(0, CHUNK)]
        pltpu.make_async_copy(a_h.at[i0], av[b], semw[b]).wait()
        if not presum:
            pltpu.make_async_copy(b_h.at[i0], bv[b], semw[b]).wait()

    def writeback(ci, b):
        o = base + ci * CHUNK
        if presum:
            def row(r, carry):
                for j in range(width // 16):
                    s = pl.ds(j * 16, 16)
                    av[b][r, s] = av[b][r, s] + bv[b][r, s]
                return carry

            lax.fori_loop(0, CHUNK, row, 0)
            pltpu.async_copy(av[b], g_h.at[pl.ds(o, CHUNK)], semw[b])
        else:
            pltpu.async_copy(av[b], gd_h.at[pl.ds(o, CHUNK)], semw[b])
            pltpu.async_copy(bv[b], gs_h.at[pl.ds(o, CHUNK)], semw[b])

    def outer(g, carry):
        for b in range(2):
            ci = g * 2 + b

            @pl.when(ci >= 2)
            def _():
                drainw(b)
            gathers(ci, b)

            @pl.when(ci >= 1)
            def _():
                drain(semg, 1 - b)
                writeback(ci - 1, 1 - b)
        return carry

    lax.fori_loop(0, NCHUNK_H // 2, outer, 0)
    drain(semg, 1)
    writeback(NCHUNK_H - 1, 1)
    for b in range(2):
        drainw(b)


def _sc_gather2(A, B, dstp, srcp, width, tc_tiling, presum, half, name):
    if presum:
        out_type = jax.ShapeDtypeStruct((EP2, width), _f32)
    else:
        out_type = (jax.ShapeDtypeStruct((EP2, width), _f32),
                    jax.ShapeDtypeStruct((EP2, width), _f32))
    scratch = [pltpu.VMEM((EPW_H,), jnp.int32),
               pltpu.VMEM((EPW_H,), jnp.int32),
               pltpu.VMEM((CHUNK, width), _f32),
               pltpu.VMEM((CHUNK, width), _f32),
               pltpu.VMEM((CHUNK, width), _f32),
               pltpu.VMEM((CHUNK, width), _f32),
               pltpu.SemaphoreType.DMA,
               pltpu.SemaphoreType.DMA,
               pltpu.SemaphoreType.DMA,
               pltpu.SemaphoreType.DMA]
    fn = pl.kernel(functools.partial(_gather2_body, width, presum, half),
                   out_type=out_type, mesh=_sc_mesh(),
                   scratch_types=scratch, name=name,
                   compiler_params=pltpu.CompilerParams(
                       use_tc_tiling_on_sc=tc_tiling))
    return fn(A, B, dstp, srcp)


def _sc_gather(A, B, XQ, dstp, srcp, half):
    G = _sc_gather2(A, B, dstp, srcp, HH, True, True, half,
                    f"sc_gather_ab{half}")
    XD, XS = _sc_gather2(XQ, XQ, dstp, srcp, 16, False, False, half,
                         f"sc_gather_x{half}")
    return G, XD, XS


def _zero_vmem(ref, nrow, ncol):
    z = jnp.zeros((16,), _f32)

    def row(r, carry):
        for j in range(ncol // 16):
            ref[r, pl.ds(j * 16, 16)] = z
        return carry

    lax.fori_loop(0, nrow, row, 0)


def _scatter_body(width, half, m2_h, dst_h, magg_h,
                  di0, di1, mv0, mv1, accm,
                  seml0, seml1, sema0, sema1):
    cid = lax.axis_index("c")
    sid = lax.axis_index("s")
    wid = sid * NC + cid
    base = wid * EPW_H
    gbase = half * EP2 + base
    row0 = sid * ROWS_PER_TILE
    di = (di0, di1)
    mv = (mv0, mv1)
    seml = (seml0, seml1)
    sema = (sema0, sema1)

    _zero_vmem(mv0, CHUNK, width)
    for j in range(ROWS_PER_TILE // CHUNK):
        pltpu.sync_copy(mv0, accm.at[pl.ds(row0 + j * CHUNK, CHUNK)])
    plsc.subcore_barrier()

    def loads(ci, b):
        o = base + ci * CHUNK
        pltpu.async_copy(dst_h.at[pl.ds(gbase + ci * CHUNK, CHUNK)],
                         di[b], seml[b])
        pltpu.async_copy(m2_h.at[pl.ds(o, CHUNK)], mv[b], seml[b])

    def adds(b):
        pltpu.async_copy(mv[b], accm.at[di[b]], sema[b], add=True)

    def drain_loads(b):
        pltpu.make_async_copy(dst_h.at[pl.ds(0, CHUNK)], di[b],
                              seml[b]).wait()
        pltpu.make_async_copy(m2_h.at[pl.ds(0, CHUNK)], mv[b],
                              seml[b]).wait()

    def drain_adds(b):
        pltpu.make_async_copy(m2_h.at[pl.ds(0, CHUNK)], mv[b],
                              sema[b]).wait()

    def outer(g, carry):
        for b in range(2):
            ci = g * 2 + b

            @pl.when(ci >= 2)
            def _():
                drain_adds(b)
            loads(ci, b)

            @pl.when(ci >= 1)
            def _():
                drain_loads(1 - b)
                adds(1 - b)
        return carry

    lax.fori_loop(0, NCHUNK_H // 2, outer, 0)
    drain_loads(1)
    adds(1)
    for b in range(2):
        drain_adds(b)
    plsc.subcore_barrier()

    for j in range(ROWS_PER_TILE // CHUNK):
        r = row0 + j * CHUNK
        pltpu.sync_copy(accm.at[pl.ds(r, CHUNK)],
                        magg_h.at[cid, pl.ds(r, CHUNK)])


def _sc_scatter1(m2, dstp, width, tc_tiling, half, name):
    out_type = jax.ShapeDtypeStruct((NC, NP, width), _f32)
    scratch = [pltpu.VMEM((CHUNK,), jnp.int32),
               pltpu.VMEM((CHUNK,), jnp.int32),
               pltpu.VMEM((CHUNK, width), _f32),
               pltpu.VMEM((CHUNK, width), _f32),
               pltpu.VMEM_SHARED((NP, width), _f32),
               pltpu.SemaphoreType.DMA,
               pltpu.SemaphoreType.DMA,
               pltpu.SemaphoreType.DMA,
               pltpu.SemaphoreType.DMA]
    fn = pl.kernel(functools.partial(_scatter_body, width, half),
                   out_type=out_type, mesh=_sc_mesh(), scratch_types=scratch,
                   name=name,
                   compiler_params=pltpu.CompilerParams(
                       use_tc_tiling_on_sc=tc_tiling))
    return fn(m2, dstp)


BE = 512
BN = 512


def _full(x):
    return pl.BlockSpec(x.shape, lambda i: (0,) * x.ndim)


def _blk(bs):
    nd = len(bs)
    return pl.BlockSpec(bs, lambda i: (i,) + (0,) * (nd - 1))


def _edge_kernel_body(coord, g, xd, xs, ea, we, w1e, b1, be_, wd2,
                      w2, b2, c1, c1b, c2r, m2_o, t_o=None):
    m_blk = jnp.dot(we[...], w1e[...], preferred_element_type=_f32)
    b1p = b1[...] + jnp.dot(be_[...], w1e[...], preferred_element_type=_f32)
    rel = xd[...] - xs[...]
    d2 = jnp.sum(rel * rel, axis=1, keepdims=True)
    pre = (g[...] + d2 * wd2[...]
           + jnp.dot(ea[...], m_blk, preferred_element_type=_f32) + b1p)
    m = _silu(pre)
    m2 = _silu(jnp.dot(m.astype(_bf16), w2[...].astype(_bf16),
                       preferred_element_type=_f32) + b2[...])
    m2_o[...] = m2
    if coord:
        u2 = _silu(jnp.dot(m2.astype(_bf16), c1[...].astype(_bf16),
                           preferred_element_type=_f32) + c1b[...])
        cw = jnp.sum(u2 * c2r[...], axis=1, keepdims=True)
        lane3 = lax.broadcasted_iota(jnp.int32, (1, 16), 1) == 3
        t_o[...] = rel * cw + lane3.astype(_f32)


def _tc_edge(coord, g, xd, xs, eap, we, w1e, b1, be_, wd2, w2, b2,
             c1, c1b, c2r):
    grid = eap.shape[0] // BE
    in_specs = [_blk((BE, HH)), _blk((BE, 16)), _blk((BE, 16)),
                _blk((BE, 16))] + [_full(w) for w in
                                   (we, w1e, b1, be_, wd2, w2, b2, c1, c1b, c2r)]
    if coord:
        out_shape = (jax.ShapeDtypeStruct((EP2, HH), _f32),
                     jax.ShapeDtypeStruct((EP2, 16), _f32))
        out_specs = (_blk((BE, HH)), _blk((BE, 16)))
    else:
        out_shape = jax.ShapeDtypeStruct((EP2, HH), _f32)
        out_specs = _blk((BE, HH))
    return pl.pallas_call(
        functools.partial(_edge_kernel_body, coord),
        grid=(grid,), in_specs=in_specs, out_specs=out_specs,
        out_shape=out_shape)(g, xd, xs, eap, we, w1e, b1, be_, wd2,
                             w2, b2, c1, c1b, c2r)


def _node_kernel_body(coord, *refs):
    if coord:
        (h, m00, m01, m10, m11, t00, t01, t10, t11, xq,
         n1a, n1b, b1n, n2, b2n, g, bb, w1d, w1s,
         h_o, xq_o, a_o, b_o) = refs
    else:
        (h, m00, m01, m10, m11, n1a, n1b, b1n, n2, b2n, g, bb, h_o) = refs
    hv = h[...]
    magg = ((m00[...][0] + m01[...][0]) + (m10[...][0] + m11[...][0]))
    u = _silu(jnp.dot(hv, n1a[...], preferred_element_type=_f32)
              + jnp.dot(magg, n1b[...], preferred_element_type=_f32) + b1n[...])
    hn = hv + jnp.dot(u, n2[...], preferred_element_type=_f32) + b2n[...]
    mu = jnp.mean(hn, axis=1, keepdims=True)
    ctr = hn - mu
    var = jnp.mean(ctr * ctr, axis=1, keepdims=True)
    hln = ctr * jax.lax.rsqrt(var + 1e-5) * g[...] + bb[...]
    h_o[...] = hln
    if coord:
        tacc = ((t00[...][0] + t01[...][0]) + (t10[...][0] + t11[...][0]))
        deg = tacc[:, 3:4]
        invd = 1.0 / jnp.maximum(deg, 1.0)
        lane = lax.broadcasted_iota(jnp.int32, (1, 16), 1)
        xq_o[...] = xq[...] + jnp.where(lane < 3, tacc, 0.0) * invd
        a_o[...] = jnp.dot(hln, w1d[...], preferred_element_type=_f32)
        b_o[...] = jnp.dot(hln, w1s[...], preferred_element_type=_f32)


def _tc_node(coord, h, magg0, magg1, tacc0, tacc1, xq, n1a, n1b, b1n,
             n2, b2n, g, bb, w1d, w1s):
    grid = NP // BN
    mspec0 = pl.BlockSpec((1, BN, HH), lambda i: (0, i, 0))
    mspec1 = pl.BlockSpec((1, BN, HH), lambda i: (1, i, 0))
    mspecs = [mspec0, mspec1, mspec0, mspec1]
    wspecs = [_full(w) for w in (n1a, n1b, b1n, n2, b2n, g, bb)]
    if coord:
        tspec0 = pl.BlockSpec((1, BN, 16), lambda i: (0, i, 0))
        tspec1 = pl.BlockSpec((1, BN, 16), lambda i: (1, i, 0))
        in_specs = [_blk((BN, HH))] + mspecs + \
                   [tspec0, tspec1, tspec0, tspec1,
                    _blk((BN, 16))] + wspecs + [_full(w1d), _full(w1s)]
        out_shape = (jax.ShapeDtypeStruct((NP, HH), _f32),
                     jax.ShapeDtypeStruct((NP, 16), _f32),
                     jax.ShapeDtypeStruct((NP, HH), _f32),
                     jax.ShapeDtypeStruct((NP, HH), _f32))
        out_specs = (_blk((BN, HH)), _blk((BN, 16)), _blk((BN, HH)),
                     _blk((BN, HH)))
        args = (h, magg0, magg0, magg1, magg1, tacc0, tacc0, tacc1, tacc1,
                xq, n1a, n1b, b1n, n2, b2n, g, bb, w1d, w1s)
    else:
        in_specs = [_blk((BN, HH))] + mspecs + wspecs
        out_shape = jax.ShapeDtypeStruct((NP, HH), _f32)
        out_specs = _blk((BN, HH))
        args = (h, magg0, magg0, magg1, magg1, n1a, n1b, b1n, n2, b2n, g, bb)
    return pl.pallas_call(
        functools.partial(_node_kernel_body, coord),
        grid=(grid,), in_specs=in_specs, out_specs=out_specs,
        out_shape=out_shape)(*args)


def _init_kernel_body(nf, pos, wn, bn, w1d, w1s, h_o, xq_o, a_o, b_o):
    h = (jnp.dot(jnp.clip(nf[...], -100.0, 100.0), wn[...],
                 preferred_element_type=_f32) + bn[...])
    h_o[...] = h
    xq_o[...] = jnp.clip(pos[...], -500.0, 500.0)
    a_o[...] = jnp.dot(h, w1d[...], preferred_element_type=_f32)
    b_o[...] = jnp.dot(h, w1s[...], preferred_element_type=_f32)


def _tc_init(nfp, pos16, wn, bn, w1d, w1s):
    grid = NP // BN
    in_specs = [_blk((BN, 128)), _blk((BN, 16))] + \
               [_full(w) for w in (wn, bn, w1d, w1s)]
    out_shape = (jax.ShapeDtypeStruct((NP, HH), _f32),
                 jax.ShapeDtypeStruct((NP, 16), _f32),
                 jax.ShapeDtypeStruct((NP, HH), _f32),
                 jax.ShapeDtypeStruct((NP, HH), _f32))
    out_specs = (_blk((BN, HH)), _blk((BN, 16)), _blk((BN, HH)),
                 _blk((BN, HH)))
    return pl.pallas_call(
        _init_kernel_body, grid=(grid,), in_specs=in_specs,
        out_specs=out_specs, out_shape=out_shape)(nfp, pos16, wn, bn,
                                                  w1d, w1s)


def kernel(node_features, positions, edge_index, edge_attr, params):
    src = edge_index[0]
    dst = edge_index[1]
    dstp = jnp.pad(dst, (0, EP - EE), constant_values=NN)
    srcp = jnp.pad(src, (0, EP - EE), constant_values=NN)
    nfp = jnp.pad(node_features, ((0, NP - NN), (0, 0)))
    pos16 = jnp.pad(positions, ((0, NP - NN), (0, 13)))

    we = params["edge_embed"]["W"]
    be_ = params["edge_embed"]["b"][None]
    lw = []
    for lp in params["layers"]:
        w1 = lp["edge1"]["W"]
        lw.append(dict(
            w1d=w1[:HH], w1s=w1[HH:2 * HH], wd2=w1[2 * HH:2 * HH + 1],
            w1e=w1[2 * HH + 1:], b1=lp["edge1"]["b"][None],
            w2=lp["edge2"]["W"], b2=lp["edge2"]["b"][None],
            c1=lp["coord1"]["W"], c1b=lp["coord1"]["b"][None],
            c2r=lp["coord2"]["W"].T,
            n1a=lp["node1"]["W"][:HH], n1b=lp["node1"]["W"][HH:],
            b1n=lp["node1"]["b"][None], n2=lp["node2"]["W"],
            b2n=lp["node2"]["b"][None], g=lp["ln_g"][None],
            bb=lp["ln_b"][None]))

    h, xq, A, B = _tc_init(nfp, pos16, params["node_embed"]["W"],
                           params["node_embed"]["b"][None],
                           lw[0]["w1d"], lw[0]["w1s"])

    ea_half = (edge_attr[:EP2], edge_attr[EP2:])
    for i in range(NLAYER):
        w = lw[i]
        coord = i < NLAYER - 1
        Maggs, Taccs = [], []
        for hf in range(NHALF):
            G, XD, XS = _sc_gather(A, B, xq, dstp, srcp, hf)
            if coord:
                m2, T = _tc_edge(True, G, XD, XS, ea_half[hf], we,
                                 w["w1e"], w["b1"], be_, w["wd2"], w["w2"],
                                 w["b2"], w["c1"], w["c1b"], w["c2r"])
                Taccs.append(_sc_scatter1(T, dstp, 16, False, hf,
                                          f"sc_scatter_t{hf}"))
            else:
                m2 = _tc_edge(False, G, XD, XS, ea_half[hf], we,
                              w["w1e"], w["b1"], be_, w["wd2"], w["w2"],
                              w["b2"], w["c1"], w["c1b"], w["c2r"])
            Maggs.append(_sc_scatter1(m2, dstp, HH, True, hf,
                                      f"sc_scatter_m{hf}"))
        if coord:
            nx = lw[i + 1]
            h, xq, A, B = _tc_node(
                True, h, Maggs[0], Maggs[1], Taccs[0], Taccs[1], xq,
                w["n1a"], w["n1b"], w["b1n"], w["n2"], w["b2n"],
                w["g"], w["bb"], nx["w1d"], nx["w1s"])
        else:
            h = _tc_node(False, h, Maggs[0], Maggs[1], None, None, None,
                         w["n1a"], w["n1b"], w["b1n"], w["n2"], w["b2n"],
                         w["g"], w["bb"], None, None)
    return h[:NN]
